# baseline (speedup 1.0000x reference)
import functools

import jax
import jax.numpy as jnp
from jax.experimental import pallas as pl
from jax.experimental.pallas import tpu as pltpu

_TOP_K = 8
_NEG_INF = float("-inf")


def _router_kernel(x_ref, w_ref, gates_ref, idx_ref, loss_ref,
                   gsum_ref, cnt_ref, *, n_tokens, n_blocks, n_experts):
    i = pl.program_id(0)
    x = x_ref[...]
    w = w_ref[...]
    logits_t = jax.lax.dot_general(
        w, x, (((1,), (1,)), ((), ())),
        preferred_element_type=jnp.float32)

    r = logits_t.shape[1]
    e_iota = jax.lax.broadcasted_iota(jnp.int32, (n_experts, r), 0)

    work = logits_t
    idx_rows = []
    top1 = None
    for k in range(_TOP_K):
        m = jnp.max(work, axis=0, keepdims=True)
        if top1 is None:
            top1 = m
        amax = jnp.min(jnp.where(work == m, e_iota, n_experts),
                       axis=0, keepdims=True)
        idx_rows.append(amax)
        work = jnp.where(e_iota == amax, _NEG_INF, work)

    sel = work == _NEG_INF
    e = jnp.where(sel, jnp.exp(logits_t - top1), 0.0)
    denom = jnp.sum(e, axis=0, keepdims=True)
    gates_t = e / denom
    gates_ref[...] = gates_t.T
    idx_ref[...] = jnp.concatenate(idx_rows, axis=0).T

    part_g = jnp.sum(gates_t, axis=1, keepdims=True)
    part_c = jnp.sum(sel.astype(jnp.float32), axis=1, keepdims=True)

    @pl.when(i == 0)
    def _init():
        gsum_ref[...] = jnp.zeros_like(gsum_ref)
        cnt_ref[...] = jnp.zeros_like(cnt_ref)

    gsum_ref[...] += part_g
    cnt_ref[...] += part_c

    @pl.when(i == n_blocks - 1)
    def _finalize():
        inv_n = 1.0 / float(n_tokens)
        loss = jnp.sum(gsum_ref[...] * inv_n * cnt_ref[...] * inv_n)
        loss_ref[0, 0] = loss * float(n_experts)


def kernel(x, W):
    n_tokens, d_model = x.shape
    n_experts = W.shape[0]
    block_r = 512
    n_blocks = n_tokens // block_r

    grid_spec = pltpu.PrefetchScalarGridSpec(
        num_scalar_prefetch=0,
        grid=(n_blocks,),
        in_specs=[
            pl.BlockSpec((block_r, d_model), lambda i: (i, 0)),
            pl.BlockSpec((n_experts, d_model), lambda i: (0, 0)),
        ],
        out_specs=[
            pl.BlockSpec((block_r, n_experts), lambda i: (i, 0)),
            pl.BlockSpec((block_r, _TOP_K), lambda i: (i, 0)),
            pl.BlockSpec(memory_space=pltpu.SMEM),
        ],
        scratch_shapes=[
            pltpu.VMEM((n_experts, 1), jnp.float32),
            pltpu.VMEM((n_experts, 1), jnp.float32),
        ],
    )

    gates, idx, loss = pl.pallas_call(
        functools.partial(_router_kernel, n_tokens=n_tokens,
                          n_blocks=n_blocks, n_experts=n_experts),
        grid_spec=grid_spec,
        out_shape=[
            jax.ShapeDtypeStruct((n_tokens, n_experts), jnp.float32),
            jax.ShapeDtypeStruct((n_tokens, _TOP_K), jnp.int32),
            jax.ShapeDtypeStruct((1, 1), jnp.float32),
        ],
        compiler_params=pltpu.CompilerParams(
            dimension_semantics=("arbitrary",),
        ),
    )(x, W)
    return gates, idx, loss[0, 0]


# speedup vs baseline: 1.0999x; 1.0999x over previous
import functools

import jax
import jax.numpy as jnp
from jax.experimental import pallas as pl
from jax.experimental.pallas import tpu as pltpu

_TOP_K = 8
_NEG_INF = float("-inf")


def _router_kernel(x_ref, w_ref, gates_ref, idx_ref, loss_ref,
                   gsum_ref, cnt_ref, *, n_tokens, n_blocks, n_experts):
    i = pl.program_id(0)
    x = x_ref[...]
    w = w_ref[...]
    logits_t = jax.lax.dot_general(
        w, x, (((1,), (1,)), ((), ())),
        preferred_element_type=jnp.float32)

    r = logits_t.shape[1]
    e_iota = jax.lax.broadcasted_iota(jnp.int32, (n_experts, r), 0)

    work = logits_t
    idx_rows = []
    top1 = None
    for k in range(_TOP_K):
        m = jnp.max(work, axis=0, keepdims=True)
        if top1 is None:
            top1 = m
        amax = jnp.min(jnp.where(work == m, e_iota, n_experts),
                       axis=0, keepdims=True)
        idx_rows.append(amax)
        work = jnp.where(e_iota == amax, _NEG_INF, work)

    sel = work == _NEG_INF
    e = jnp.where(sel, jnp.exp(logits_t - top1), 0.0)
    denom = jnp.sum(e, axis=0, keepdims=True)
    gates_t = e / denom
    gates_ref[...] = gates_t.T
    idx_ref[...] = jnp.concatenate(idx_rows, axis=0).T

    part_g = jnp.sum(gates_t, axis=1, keepdims=True)
    part_c = jnp.sum(sel.astype(jnp.float32), axis=1, keepdims=True)

    @pl.when(i == 0)
    def _init():
        gsum_ref[...] = jnp.zeros_like(gsum_ref)
        cnt_ref[...] = jnp.zeros_like(cnt_ref)

    gsum_ref[...] += part_g
    cnt_ref[...] += part_c

    @pl.when(i == n_blocks - 1)
    def _finalize():
        inv_n = 1.0 / float(n_tokens)
        loss = jnp.sum(gsum_ref[...] * inv_n * cnt_ref[...] * inv_n)
        loss_ref[0, 0] = loss * float(n_experts)


def kernel(x, W):
    n_tokens, d_model = x.shape
    n_experts = W.shape[0]
    block_r = 1024
    n_blocks = n_tokens // block_r

    grid_spec = pltpu.PrefetchScalarGridSpec(
        num_scalar_prefetch=0,
        grid=(n_blocks,),
        in_specs=[
            pl.BlockSpec((block_r, d_model), lambda i: (i, 0)),
            pl.BlockSpec((n_experts, d_model), lambda i: (0, 0)),
        ],
        out_specs=[
            pl.BlockSpec((block_r, n_experts), lambda i: (i, 0)),
            pl.BlockSpec((block_r, _TOP_K), lambda i: (i, 0)),
            pl.BlockSpec(memory_space=pltpu.SMEM),
        ],
        scratch_shapes=[
            pltpu.VMEM((n_experts, 1), jnp.float32),
            pltpu.VMEM((n_experts, 1), jnp.float32),
        ],
    )

    gates, idx, loss = pl.pallas_call(
        functools.partial(_router_kernel, n_tokens=n_tokens,
                          n_blocks=n_blocks, n_experts=n_experts),
        grid_spec=grid_spec,
        out_shape=[
            jax.ShapeDtypeStruct((n_tokens, n_experts), jnp.float32),
            jax.ShapeDtypeStruct((n_tokens, _TOP_K), jnp.int32),
            jax.ShapeDtypeStruct((1, 1), jnp.float32),
        ],
        compiler_params=pltpu.CompilerParams(
            dimension_semantics=("arbitrary",),
        ),
    )(x, W)
    return gates, idx, loss[0, 0]
